# baseline (speedup 1.0000x reference)
import jax
import jax.numpy as jnp
from jax import lax
from jax.experimental import pallas as pl
from jax.experimental.pallas import tpu as pltpu
from jax.experimental.pallas import tpu_sc as plsc

NC, NS = 2, 16
NW = NC * NS
N = 5000
NPAD = 5120
RPT = NPAD // NS
E = 640000
EPW = E // NW
KE = 80
NCH = EPW // KE
L = 100000
LPAD = 100352
LPW = LPAD // NW
KL = 112
NCL = LPW // KL
D = 128
_f32 = jnp.float32

_mesh = plsc.VectorSubcoreMesh(
    core_axis_name="c", subcore_axis_name="s", num_cores=NC, num_subcores=NS)


def _worker_id():
    return lax.axis_index("s") * NC + lax.axis_index("c")


EPT = E // NS
SEGS = tuple((i * 7680, 7680) for i in range(5)) + ((38400, 960),)
TAIL = (39360, 640)
SEGMAX = 7680
MC = 2 * KE
NSET = 3


def _make_segsum_body(with_counts):
    def body(*refs):
        if with_counts:
            ins = refs[:7]
            (tab_u, tab_m, iu_hbm, im_hbm, zb, zs, ones_hbm) = ins
            (out_m, out_u, cnt_m, cnt_u) = refs[7:11]
            rest = refs[11:]
        else:
            ins = refs[:5]
            (tab_u, tab_m, iu_hbm, im_hbm, zb) = ins
            (out_m, out_u) = refs[5:7]
            rest = refs[7:]
        k = 0

        def take(n):
            nonlocal k
            r = rest[k:k + n]
            k += n
            return r

        acc, = take(1)
        cnt = take(1)[0] if with_counts else None
        gi_all, si_all = take(2)
        gi_f = take(2 * NSET)
        si_f = take(2 * NSET)
        r_f = take(2 * NSET)
        gi_w = tuple(gi_f[2 * b:2 * b + 2] for b in range(NSET))
        si_w = tuple(si_f[2 * b:2 * b + 2] for b in range(NSET))
        rw = tuple(r_f[2 * b:2 * b + 2] for b in range(NSET))
        if with_counts:
            ones_v, cv = take(2)
        s_g = take(NSET)
        s_s = take(NSET)
        if with_counts:
            s_c = take(NSET)

        cid = lax.axis_index("c")
        tid = lax.axis_index("s")
        r0 = tid * RPT
        pltpu.sync_copy(zb.at[pl.ds(r0, RPT)], acc.at[pl.ds(r0, RPT)])
        if with_counts:
            pltpu.sync_copy(zs.at[pl.ds(r0, RPT)], cv)
            pltpu.sync_copy(cv, cnt.at[pl.ds(r0, RPT)])
            pltpu.sync_copy(ones_hbm, ones_v)

        def run_dir(tab, gidx_hbm, sidx_hbm, out_a, cnt_o):
            def step(c, b, first):
                if not first:
                    for sub in (0, 1):
                        pltpu.make_async_copy(
                            rw[b][sub], acc.at[si_w[b][sub]], s_s[b]).wait()
                        if with_counts:
                            pltpu.make_async_copy(
                                ones_v, cnt.at[si_w[b][sub]], s_c[b]).wait()
                for sub in (0, 1):
                    giv, siv = gi_w[b][sub], si_w[b][sub]
                    off = c * MC + sub * KE
                    for i in range(KE // 16):
                        giv[pl.ds(i * 16, 16)] = gi_all[pl.ds(off + i * 16, 16)]
                        siv[pl.ds(i * 16, 16)] = si_all[pl.ds(off + i * 16, 16)]
                d = [pltpu.async_copy(tab.at[gi_w[b][sub]], rw[b][sub], s_g[b])
                     for sub in (0, 1)]
                for sub in (0, 1):
                    d[sub].wait()
                    pltpu.async_copy(rw[b][sub], acc.at[si_w[b][sub]],
                                     s_s[b], add=True)
                    if with_counts:
                        pltpu.async_copy(ones_v, cnt.at[si_w[b][sub]],
                                         s_c[b], add=True)

            first_seg = True
            for soff, slen in SEGS:
                base = tid * EPT + soff
                pltpu.sync_copy(gidx_hbm.at[pl.ds(base, slen)],
                                gi_all.at[pl.ds(0, slen)])
                pltpu.sync_copy(sidx_hbm.at[pl.ds(base, slen)],
                                si_all.at[pl.ds(0, slen)])
                if first_seg:
                    plsc.subcore_barrier()
                for b in range(NSET):
                    step(b, b, first_seg)

                def quad(p, carry):
                    for b in range(NSET):
                        step(NSET * p + b, b, False)
                    return carry

                lax.fori_loop(1, slen // MC // NSET, quad, 0)
                first_seg = False
            toff, tlen = TAIL
            pltpu.sync_copy(gidx_hbm.at[pl.ds(tid * EPT + toff, tlen)],
                            gi_all.at[pl.ds(0, tlen)])
            pltpu.sync_copy(sidx_hbm.at[pl.ds(tid * EPT + toff, tlen)],
                            si_all.at[pl.ds(0, tlen)])
            for i, b in enumerate((0, 1, 2, 0)):
                step(i, b, False)
            for b in range(NSET):
                for sub in (0, 1):
                    pltpu.make_async_copy(
                        rw[b][sub], acc.at[si_w[b][sub]], s_s[b]).wait()
                    if with_counts:
                        pltpu.make_async_copy(
                            ones_v, cnt.at[si_w[b][sub]], s_c[b]).wait()
            plsc.subcore_barrier()
            pltpu.sync_copy(acc.at[pl.ds(r0, RPT)], out_a.at[pl.ds(r0, RPT)])
            if with_counts:
                pltpu.sync_copy(cnt.at[pl.ds(r0, RPT)], cv)
                pltpu.sync_copy(cv, cnt_o.at[pl.ds(r0, RPT)])

        @pl.when(cid == 0)
        def _():
            run_dir(tab_u, iu_hbm, im_hbm, out_m, cnt_m if with_counts else None)

        @pl.when(cid == 1)
        def _():
            run_dir(tab_m, im_hbm, iu_hbm, out_u, cnt_u if with_counts else None)

    return body


def _seg_scratch(with_counts):
    sc = [
        pltpu.VMEM_SHARED((NPAD, D), _f32),
        pltpu.VMEM_SHARED((NPAD,), _f32) if with_counts else None,
        pltpu.VMEM((SEGMAX,), jnp.int32),
        pltpu.VMEM((SEGMAX,), jnp.int32),
    ]
    sc += [pltpu.VMEM((KE,), jnp.int32) for _ in range(4 * NSET)]
    sc += [pltpu.VMEM((KE, D), _f32) for _ in range(2 * NSET)]
    if with_counts:
        sc += [pltpu.VMEM((KE,), _f32), pltpu.VMEM((RPT,), _f32)]
    sc = [s for s in sc if s is not None]
    nsem = 3 * NSET if with_counts else 2 * NSET
    sc += [pltpu.SemaphoreType.DMA] * nsem
    return sc


_segsum_counts = pl.kernel(
    _make_segsum_body(True),
    out_type=(jax.ShapeDtypeStruct((NPAD, D), _f32),
              jax.ShapeDtypeStruct((NPAD, D), _f32),
              jax.ShapeDtypeStruct((NPAD,), _f32),
              jax.ShapeDtypeStruct((NPAD,), _f32)),
    mesh=_mesh,
    scratch_types=_seg_scratch(True),
)

_segsum = pl.kernel(
    _make_segsum_body(False),
    out_type=(jax.ShapeDtypeStruct((NPAD, D), _f32),
              jax.ShapeDtypeStruct((NPAD, D), _f32)),
    mesh=_mesh,
    scratch_types=_seg_scratch(False),
)


def _label_gather_body(pu_hbm, pm_hbm, row_hbm, col_hbm,
                       gu_o, gm_o, ri_all, ci_all,
                       ru0, ru1, rm0, rm1, sgu0, sgu1, sgm0, sgm1,
                       swu0, swu1, swm0, swm1):
    ru_w, rm_w = (ru0, ru1), (rm0, rm1)
    s_g1, s_g2 = (sgu0, sgu1), (sgm0, sgm1)
    s_w1, s_w2 = (swu0, swu1), (swm0, swm1)
    base = _worker_id() * LPW
    pltpu.sync_copy(row_hbm.at[pl.ds(base, LPW)], ri_all)
    pltpu.sync_copy(col_hbm.at[pl.ds(base, LPW)], ci_all)

    def step(c, b, first):
        ru, rm = ru_w[b], rm_w[b]
        if not first:
            off2 = base + (c - 2) * KL
            pltpu.make_async_copy(ru, gu_o.at[pl.ds(off2, KL)], s_w1[b]).wait()
            pltpu.make_async_copy(rm, gm_o.at[pl.ds(off2, KL)], s_w2[b]).wait()
        loc = c * KL
        d1 = pltpu.async_copy(pu_hbm.at[ri_all.at[pl.ds(loc, KL)]], ru, s_g1[b])
        d2 = pltpu.async_copy(pm_hbm.at[ci_all.at[pl.ds(loc, KL)]], rm, s_g2[b])
        d1.wait()
        d2.wait()
        off = base + loc
        pltpu.async_copy(ru, gu_o.at[pl.ds(off, KL)], s_w1[b])
        pltpu.async_copy(rm, gm_o.at[pl.ds(off, KL)], s_w2[b])

    step(0, 0, True)
    step(1, 1, True)

    def pair(p, carry):
        step(2 * p, 0, False)
        step(2 * p + 1, 1, False)
        return carry

    lax.fori_loop(1, NCL // 2, pair, 0)
    for b in (0, 1):
        off2 = base + (NCL - 2 + b) * KL
        pltpu.make_async_copy(ru_w[b], gu_o.at[pl.ds(off2, KL)], s_w1[b]).wait()
        pltpu.make_async_copy(rm_w[b], gm_o.at[pl.ds(off2, KL)], s_w2[b]).wait()


_label_gather = pl.kernel(
    _label_gather_body,
    out_type=(jax.ShapeDtypeStruct((LPAD, D), _f32),
              jax.ShapeDtypeStruct((LPAD, D), _f32)),
    mesh=_mesh,
    scratch_types=[
        pltpu.VMEM((LPW,), jnp.int32),
        pltpu.VMEM((LPW,), jnp.int32),
        pltpu.VMEM((KL, D), _f32),
        pltpu.VMEM((KL, D), _f32),
        pltpu.VMEM((KL, D), _f32),
        pltpu.VMEM((KL, D), _f32),
    ] + [pltpu.SemaphoreType.DMA] * 8,
)


_BR = 512


def _sage_half(acc, cnt, x, Wl, b, Wr):
    s = acc[...]
    ctot = cnt[...]
    inv = 1.0 / jnp.maximum(ctot, 1.0)
    mean = s * inv[:, None]
    return (jnp.dot(mean, Wl[...], preferred_element_type=jnp.float32)
            + b[...]
            + jnp.dot(x[...], Wr[...], preferred_element_type=jnp.float32))


def _tc_layer1_body(acc_m, cnt_m, xm, WlA, bA, WrA,
                    acc_u, cnt_u, xu, WlB, bB, WrB, hm_o, hu_o):
    hm_o[...] = jnp.maximum(_sage_half(acc_m, cnt_m, xm, WlA, bA, WrA), 0.0)
    hu_o[...] = jnp.maximum(_sage_half(acc_u, cnt_u, xu, WlB, bB, WrB), 0.0)


def _tc_layer2_body(acc_m, cnt_m, xm, WlA, bA, WrA,
                    acc_u, cnt_u, xu, WlB, bB, WrB,
                    WpU, bpU, WpM, pu_o, pm_o):
    zm = _sage_half(acc_m, cnt_m, xm, WlA, bA, WrA)
    zu = _sage_half(acc_u, cnt_u, xu, WlB, bB, WrB)
    pu_o[...] = jnp.dot(zu, WpU[...], preferred_element_type=jnp.float32) + bpU[...]
    pm_o[...] = jnp.dot(zm, WpM[...], preferred_element_type=jnp.float32)


def _acc_spec():
    return pl.BlockSpec((_BR, D), lambda i: (i, 0))


def _cnt_spec():
    return pl.BlockSpec((_BR,), lambda i: (i,))


def _row_spec(w=D):
    return pl.BlockSpec((_BR, w), lambda i: (i, 0))


def _w_spec(w=D):
    return pl.BlockSpec((D, w), lambda i: (0, 0))


def _b_spec(w=D):
    return pl.BlockSpec((1, w), lambda i: (0, 0))


_tc_layer1 = pl.pallas_call(
    _tc_layer1_body,
    grid=(NPAD // _BR,),
    in_specs=[_acc_spec(), _cnt_spec(), _row_spec(), _w_spec(), _b_spec(), _w_spec(),
              _acc_spec(), _cnt_spec(), _row_spec(), _w_spec(), _b_spec(), _w_spec()],
    out_specs=(_row_spec(), _row_spec()),
    out_shape=(jax.ShapeDtypeStruct((NPAD, D), _f32),
               jax.ShapeDtypeStruct((NPAD, D), _f32)),
)

_tc_layer2 = pl.pallas_call(
    _tc_layer2_body,
    grid=(NPAD // _BR,),
    in_specs=[_acc_spec(), _cnt_spec(), _row_spec(), _w_spec(), _b_spec(), _w_spec(),
              _acc_spec(), _cnt_spec(), _row_spec(), _w_spec(), _b_spec(), _w_spec(),
              _w_spec(D), _b_spec(D), _w_spec(D)],
    out_specs=(_row_spec(D), _row_spec(D)),
    out_shape=(jax.ShapeDtypeStruct((NPAD, D), _f32),
               jax.ShapeDtypeStruct((NPAD, D), _f32)),
)


def _head_body(gu, gm, o):
    s = gu[...] + gm[...]
    colid = lax.broadcasted_iota(jnp.int32, s.shape, 1)
    o[...] = jnp.where(colid == 1, jax.nn.softplus(s) + 1e-6, s)


_BL = 1024
_head = pl.pallas_call(
    _head_body,
    grid=(LPAD // _BL,),
    in_specs=[pl.BlockSpec((_BL, D), lambda i: (i, 0)),
              pl.BlockSpec((_BL, D), lambda i: (i, 0))],
    out_specs=pl.BlockSpec((_BL, D), lambda i: (i, 0)),
    out_shape=jax.ShapeDtypeStruct((LPAD, D), _f32),
)


def kernel(x_user, x_movie, edge_index, edge_label_index,
           W1l_um, b1l_um, W1r_um, W1l_mu, b1l_mu, W1r_mu,
           W2l_um, b2l_um, W2r_um, W2l_mu, b2l_mu, W2r_mu, Wlin, blin):
    f32 = jnp.float32
    pad_n = NPAD - N
    xu = jnp.pad(x_user.astype(f32), ((0, pad_n), (0, 0)))
    xm = jnp.pad(x_movie.astype(f32), ((0, pad_n), (0, 0)))
    iu = edge_index[0].astype(jnp.int32)
    im = edge_index[1].astype(jnp.int32)
    row = jnp.pad(edge_label_index[0].astype(jnp.int32), (0, LPAD - L))
    col = jnp.pad(edge_label_index[1].astype(jnp.int32), (0, LPAD - L))

    zb = jnp.zeros((NPAD, D), f32)
    zs = jnp.zeros((NPAD,), f32)
    ones = jnp.ones((KE,), f32)

    acc_m, acc_u, cnt_m, cnt_u = _segsum_counts(xu, xm, iu, im, zb, zs, ones)

    b1um = b1l_um.reshape(1, D).astype(f32)
    b1mu = b1l_mu.reshape(1, D).astype(f32)
    h_movie, h_user = _tc_layer1(acc_m, cnt_m, xm, W1l_um, b1um, W1r_um,
                                 acc_u, cnt_u, xu, W1l_mu, b1mu, W1r_mu)

    acc2_m, acc2_u = _segsum(h_user, h_movie, iu, im, zb)

    WpU = jnp.zeros((D, D), f32).at[:, 0:2].set(Wlin[:D].astype(f32))
    WpM = jnp.zeros((D, D), f32).at[:, 0:2].set(Wlin[D:].astype(f32))
    bp = jnp.zeros((1, D), f32).at[0, 0:2].set(blin.astype(f32))
    b2um = b2l_um.reshape(1, D).astype(f32)
    b2mu = b2l_mu.reshape(1, D).astype(f32)
    p_user, p_movie = _tc_layer2(acc2_m, cnt_m, h_movie, W2l_um, b2um, W2r_um,
                                 acc2_u, cnt_u, h_user, W2l_mu, b2mu, W2r_mu,
                                 WpU, bp, WpM)

    gu, gm = _label_gather(p_user, p_movie, row, col)
    out = _head(gu, gm)
    return out[:L, 0], out[:L, 1]


# speedup vs baseline: 1.0009x; 1.0009x over previous
import jax
import jax.numpy as jnp
from jax import lax
from jax.experimental import pallas as pl
from jax.experimental.pallas import tpu as pltpu
from jax.experimental.pallas import tpu_sc as plsc

NC, NS = 2, 16
NW = NC * NS
N = 5000
NPAD = 5120
RPT = NPAD // NS
E = 640000
EPW = E // NW
KE = 80
NCH = EPW // KE
L = 100000
LPAD = 100352
LPW = LPAD // NW
KL = 112
NCL = LPW // KL
D = 128
CW = 16
_f32 = jnp.float32

_mesh = plsc.VectorSubcoreMesh(
    core_axis_name="c", subcore_axis_name="s", num_cores=NC, num_subcores=NS)


def _worker_id():
    return lax.axis_index("s") * NC + lax.axis_index("c")


EPT = E // NS
SEGS = ((0, 16000), (16000, 16000), (32000, 7680))
TAIL = (39680, 320)
SEGMAX = 16000
MC = 2 * KE
NSET = 2


def _make_segsum_body(with_counts):
    def body(*refs):
        if with_counts:
            ins = refs[:7]
            (tab_u, tab_m, iu_hbm, im_hbm, zb, zs, ones_hbm) = ins
            (out_m, out_u, cnt_m, cnt_u) = refs[7:11]
            rest = refs[11:]
        else:
            ins = refs[:5]
            (tab_u, tab_m, iu_hbm, im_hbm, zb) = ins
            (out_m, out_u) = refs[5:7]
            rest = refs[7:]
        k = 0

        def take(n):
            nonlocal k
            r = rest[k:k + n]
            k += n
            return r

        acc, = take(1)
        cnt = take(1)[0] if with_counts else None
        gi_all, si_all = take(2)
        gi_f = take(2 * NSET)
        si_f = take(2 * NSET)
        r_f = take(2 * NSET)
        gi_w = tuple(gi_f[2 * b:2 * b + 2] for b in range(NSET))
        si_w = tuple(si_f[2 * b:2 * b + 2] for b in range(NSET))
        rw = tuple(r_f[2 * b:2 * b + 2] for b in range(NSET))
        if with_counts:
            ones_v, cv = take(2)
        s_g = take(NSET)
        s_s = take(NSET)
        if with_counts:
            s_c = take(NSET)

        cid = lax.axis_index("c")
        tid = lax.axis_index("s")
        r0 = tid * RPT
        pltpu.sync_copy(zb.at[pl.ds(r0, RPT)], acc.at[pl.ds(r0, RPT)])
        if with_counts:
            pltpu.sync_copy(zs.at[pl.ds(r0, RPT)], cv)
            pltpu.sync_copy(cv, cnt.at[pl.ds(r0, RPT)])
            pltpu.sync_copy(ones_hbm, ones_v)

        def run_dir(tab, gidx_hbm, sidx_hbm, out_a, cnt_o):
            def step(c, b, first):
                if not first:
                    for sub in (0, 1):
                        pltpu.make_async_copy(
                            rw[b][sub], acc.at[si_w[b][sub]], s_s[b]).wait()
                        if with_counts:
                            pltpu.make_async_copy(
                                ones_v, cnt.at[si_w[b][sub]], s_c[b]).wait()
                for sub in (0, 1):
                    giv, siv = gi_w[b][sub], si_w[b][sub]
                    off = c * MC + sub * KE
                    for i in range(KE // 16):
                        giv[pl.ds(i * 16, 16)] = gi_all[pl.ds(off + i * 16, 16)]
                        siv[pl.ds(i * 16, 16)] = si_all[pl.ds(off + i * 16, 16)]
                d = [pltpu.async_copy(tab.at[gi_w[b][sub]], rw[b][sub], s_g[b])
                     for sub in (0, 1)]
                for sub in (0, 1):
                    d[sub].wait()
                    pltpu.async_copy(rw[b][sub], acc.at[si_w[b][sub]],
                                     s_s[b], add=True)
                    if with_counts:
                        pltpu.async_copy(ones_v, cnt.at[si_w[b][sub]],
                                         s_c[b], add=True)

            first_seg = True
            for soff, slen in SEGS:
                base = tid * EPT + soff
                pltpu.sync_copy(gidx_hbm.at[pl.ds(base, slen)],
                                gi_all.at[pl.ds(0, slen)])
                pltpu.sync_copy(sidx_hbm.at[pl.ds(base, slen)],
                                si_all.at[pl.ds(0, slen)])
                if first_seg:
                    plsc.subcore_barrier()
                for b in range(NSET):
                    step(b, b, first_seg)

                def quad(p, carry):
                    for b in range(NSET):
                        step(NSET * p + b, b, False)
                    return carry

                lax.fori_loop(1, slen // MC // NSET, quad, 0)
                first_seg = False
            toff, tlen = TAIL
            pltpu.sync_copy(gidx_hbm.at[pl.ds(tid * EPT + toff, tlen)],
                            gi_all.at[pl.ds(0, tlen)])
            pltpu.sync_copy(sidx_hbm.at[pl.ds(tid * EPT + toff, tlen)],
                            si_all.at[pl.ds(0, tlen)])
            for i, b in enumerate((0, 1)):
                step(i, b, False)
            for b in range(NSET):
                for sub in (0, 1):
                    pltpu.make_async_copy(
                        rw[b][sub], acc.at[si_w[b][sub]], s_s[b]).wait()
                    if with_counts:
                        pltpu.make_async_copy(
                            ones_v, cnt.at[si_w[b][sub]], s_c[b]).wait()
            plsc.subcore_barrier()
            pltpu.sync_copy(acc.at[pl.ds(r0, RPT)], out_a.at[pl.ds(r0, RPT)])
            if with_counts:
                pltpu.sync_copy(cnt.at[pl.ds(r0, RPT)], cv)
                pltpu.sync_copy(cv, cnt_o.at[pl.ds(r0, RPT)])

        @pl.when(cid == 0)
        def _():
            run_dir(tab_u, iu_hbm, im_hbm, out_m, cnt_m if with_counts else None)

        @pl.when(cid == 1)
        def _():
            run_dir(tab_m, im_hbm, iu_hbm, out_u, cnt_u if with_counts else None)

    return body


def _seg_scratch(with_counts):
    sc = [
        pltpu.VMEM_SHARED((NPAD, D), _f32),
        pltpu.VMEM_SHARED((NPAD,), _f32) if with_counts else None,
        pltpu.VMEM((SEGMAX,), jnp.int32),
        pltpu.VMEM((SEGMAX,), jnp.int32),
    ]
    sc += [pltpu.VMEM((KE,), jnp.int32) for _ in range(4 * NSET)]
    sc += [pltpu.VMEM((KE, D), _f32) for _ in range(2 * NSET)]
    if with_counts:
        sc += [pltpu.VMEM((KE,), _f32), pltpu.VMEM((RPT,), _f32)]
    sc = [s for s in sc if s is not None]
    nsem = 3 * NSET if with_counts else 2 * NSET
    sc += [pltpu.SemaphoreType.DMA] * nsem
    return sc


_segsum_counts = pl.kernel(
    _make_segsum_body(True),
    out_type=(jax.ShapeDtypeStruct((NPAD, D), _f32),
              jax.ShapeDtypeStruct((NPAD, D), _f32),
              jax.ShapeDtypeStruct((NPAD,), _f32),
              jax.ShapeDtypeStruct((NPAD,), _f32)),
    mesh=_mesh,
    scratch_types=_seg_scratch(True),
)

_segsum = pl.kernel(
    _make_segsum_body(False),
    out_type=(jax.ShapeDtypeStruct((NPAD, D), _f32),
              jax.ShapeDtypeStruct((NPAD, D), _f32)),
    mesh=_mesh,
    scratch_types=_seg_scratch(False),
)


def _label_gather_body(pu_hbm, pm_hbm, row_hbm, col_hbm,
                       gu_o, gm_o, ri_all, ci_all,
                       ru0, ru1, rm0, rm1,
                       sg0, sg1, sw0, sw1):
    ru_w, rm_w = (ru0, ru1), (rm0, rm1)
    s_g, s_w = (sg0, sg1), (sw0, sw1)
    base = _worker_id() * LPW
    pltpu.sync_copy(row_hbm.at[pl.ds(base, LPW)], ri_all)
    pltpu.sync_copy(col_hbm.at[pl.ds(base, LPW)], ci_all)

    def step(c, b, first):
        ru, rm = ru_w[b], rm_w[b]
        if not first:
            off2 = base + (c - 2) * KL
            pltpu.make_async_copy(ru, gu_o.at[pl.ds(off2, KL)], s_w[b]).wait()
            pltpu.make_async_copy(rm, gm_o.at[pl.ds(off2, KL)], s_w[b]).wait()
        loc = c * KL
        d1 = pltpu.async_copy(pu_hbm.at[ri_all.at[pl.ds(loc, KL)]], ru, s_g[b])
        d2 = pltpu.async_copy(pm_hbm.at[ci_all.at[pl.ds(loc, KL)]], rm, s_g[b])
        d1.wait()
        d2.wait()
        off = base + loc
        pltpu.async_copy(ru, gu_o.at[pl.ds(off, KL)], s_w[b])
        pltpu.async_copy(rm, gm_o.at[pl.ds(off, KL)], s_w[b])

    step(0, 0, True)
    step(1, 1, True)

    def pair(p, carry):
        step(2 * p, 0, False)
        step(2 * p + 1, 1, False)
        return carry

    lax.fori_loop(1, NCL // 2, pair, 0)
    for b in (0, 1):
        off2 = base + (NCL - 2 + b) * KL
        pltpu.make_async_copy(ru_w[b], gu_o.at[pl.ds(off2, KL)], s_w[b]).wait()
        pltpu.make_async_copy(rm_w[b], gm_o.at[pl.ds(off2, KL)], s_w[b]).wait()


_label_gather = pl.kernel(
    _label_gather_body,
    out_type=(jax.ShapeDtypeStruct((LPAD, D), _f32),
              jax.ShapeDtypeStruct((LPAD, D), _f32)),
    mesh=_mesh,
    scratch_types=[
        pltpu.VMEM((LPW,), jnp.int32),
        pltpu.VMEM((LPW,), jnp.int32),
        pltpu.VMEM((KL, D), _f32),
        pltpu.VMEM((KL, D), _f32),
        pltpu.VMEM((KL, D), _f32),
        pltpu.VMEM((KL, D), _f32),
    ] + [pltpu.SemaphoreType.DMA] * 4,
)


_BR = 512


def _sage_half(acc, cnt, x, Wl, b, Wr):
    s = acc[...]
    ctot = cnt[...]
    inv = 1.0 / jnp.maximum(ctot, 1.0)
    mean = s * inv[:, None]
    return (jnp.dot(mean, Wl[...], preferred_element_type=jnp.float32)
            + b[...]
            + jnp.dot(x[...], Wr[...], preferred_element_type=jnp.float32))


def _tc_layer1_body(acc_m, cnt_m, xm, WlA, bA, WrA,
                    acc_u, cnt_u, xu, WlB, bB, WrB, hm_o, hu_o):
    hm_o[...] = jnp.maximum(_sage_half(acc_m, cnt_m, xm, WlA, bA, WrA), 0.0)
    hu_o[...] = jnp.maximum(_sage_half(acc_u, cnt_u, xu, WlB, bB, WrB), 0.0)


def _tc_layer2_body(acc_m, cnt_m, xm, WlA, bA, WrA,
                    acc_u, cnt_u, xu, WlB, bB, WrB,
                    WpU, bpU, WpM, pu_o, pm_o):
    zm = _sage_half(acc_m, cnt_m, xm, WlA, bA, WrA)
    zu = _sage_half(acc_u, cnt_u, xu, WlB, bB, WrB)
    pu_o[...] = jnp.dot(zu, WpU[...], preferred_element_type=jnp.float32) + bpU[...]
    pm_o[...] = jnp.dot(zm, WpM[...], preferred_element_type=jnp.float32)


def _acc_spec():
    return pl.BlockSpec((_BR, D), lambda i: (i, 0))


def _cnt_spec():
    return pl.BlockSpec((_BR,), lambda i: (i,))


def _row_spec(w=D):
    return pl.BlockSpec((_BR, w), lambda i: (i, 0))


def _w_spec(w=D):
    return pl.BlockSpec((D, w), lambda i: (0, 0))


def _b_spec(w=D):
    return pl.BlockSpec((1, w), lambda i: (0, 0))


_tc_layer1 = pl.pallas_call(
    _tc_layer1_body,
    grid=(NPAD // _BR,),
    in_specs=[_acc_spec(), _cnt_spec(), _row_spec(), _w_spec(), _b_spec(), _w_spec(),
              _acc_spec(), _cnt_spec(), _row_spec(), _w_spec(), _b_spec(), _w_spec()],
    out_specs=(_row_spec(), _row_spec()),
    out_shape=(jax.ShapeDtypeStruct((NPAD, D), _f32),
               jax.ShapeDtypeStruct((NPAD, D), _f32)),
)

_tc_layer2 = pl.pallas_call(
    _tc_layer2_body,
    grid=(NPAD // _BR,),
    in_specs=[_acc_spec(), _cnt_spec(), _row_spec(), _w_spec(), _b_spec(), _w_spec(),
              _acc_spec(), _cnt_spec(), _row_spec(), _w_spec(), _b_spec(), _w_spec(),
              _w_spec(D), _b_spec(D), _w_spec(D)],
    out_specs=(_row_spec(D), _row_spec(D)),
    out_shape=(jax.ShapeDtypeStruct((NPAD, D), _f32),
               jax.ShapeDtypeStruct((NPAD, D), _f32)),
)


def _head_body(gu, gm, o):
    s = gu[...] + gm[...]
    colid = lax.broadcasted_iota(jnp.int32, s.shape, 1)
    o[...] = jnp.where(colid == 1, jax.nn.softplus(s) + 1e-6, s)


_BL = 1024
_head = pl.pallas_call(
    _head_body,
    grid=(LPAD // _BL,),
    in_specs=[pl.BlockSpec((_BL, D), lambda i: (i, 0)),
              pl.BlockSpec((_BL, D), lambda i: (i, 0))],
    out_specs=pl.BlockSpec((_BL, D), lambda i: (i, 0)),
    out_shape=jax.ShapeDtypeStruct((LPAD, D), _f32),
)


def kernel(x_user, x_movie, edge_index, edge_label_index,
           W1l_um, b1l_um, W1r_um, W1l_mu, b1l_mu, W1r_mu,
           W2l_um, b2l_um, W2r_um, W2l_mu, b2l_mu, W2r_mu, Wlin, blin):
    f32 = jnp.float32
    pad_n = NPAD - N
    xu = jnp.pad(x_user.astype(f32), ((0, pad_n), (0, 0)))
    xm = jnp.pad(x_movie.astype(f32), ((0, pad_n), (0, 0)))
    iu = edge_index[0].astype(jnp.int32)
    im = edge_index[1].astype(jnp.int32)
    row = jnp.pad(edge_label_index[0].astype(jnp.int32), (0, LPAD - L))
    col = jnp.pad(edge_label_index[1].astype(jnp.int32), (0, LPAD - L))

    zb = jnp.zeros((NPAD, D), f32)
    zs = jnp.zeros((NPAD,), f32)
    ones = jnp.ones((KE,), f32)

    acc_m, acc_u, cnt_m, cnt_u = _segsum_counts(xu, xm, iu, im, zb, zs, ones)

    b1um = b1l_um.reshape(1, D).astype(f32)
    b1mu = b1l_mu.reshape(1, D).astype(f32)
    h_movie, h_user = _tc_layer1(acc_m, cnt_m, xm, W1l_um, b1um, W1r_um,
                                 acc_u, cnt_u, xu, W1l_mu, b1mu, W1r_mu)

    acc2_m, acc2_u = _segsum(h_user, h_movie, iu, im, zb)

    WpU = jnp.zeros((D, D), f32).at[:, 0:2].set(Wlin[:D].astype(f32))
    WpM = jnp.zeros((D, D), f32).at[:, 0:2].set(Wlin[D:].astype(f32))
    bp = jnp.zeros((1, D), f32).at[0, 0:2].set(blin.astype(f32))
    b2um = b2l_um.reshape(1, D).astype(f32)
    b2mu = b2l_mu.reshape(1, D).astype(f32)
    p_user, p_movie = _tc_layer2(acc2_m, cnt_m, h_movie, W2l_um, b2um, W2r_um,
                                 acc2_u, cnt_u, h_user, W2l_mu, b2mu, W2r_mu,
                                 WpU, bp, WpM)

    gu, gm = _label_gather(p_user, p_movie, row, col)
    out = _head(gu, gm)
    return out[:L, 0], out[:L, 1]


# speedup vs baseline: 1.0020x; 1.0011x over previous
import jax
import jax.numpy as jnp
from jax import lax
from jax.experimental import pallas as pl
from jax.experimental.pallas import tpu as pltpu
from jax.experimental.pallas import tpu_sc as plsc

NC, NS = 2, 16
NW = NC * NS
N = 5000
NPAD = 5120
RPT = NPAD // NS
E = 640000
EPW = E // NW
KE = 80
NCH = EPW // KE
L = 100000
LPAD = 100352
LPW = LPAD // NW
KL = 112
NCL = LPW // KL
D = 128
CW = 16
_f32 = jnp.float32

_mesh = plsc.VectorSubcoreMesh(
    core_axis_name="c", subcore_axis_name="s", num_cores=NC, num_subcores=NS)


def _worker_id():
    return lax.axis_index("s") * NC + lax.axis_index("c")


EPT = E // NS
SEGS = ((0, 16000), (16000, 16000), (32000, 8000))
TAIL = (0, 0)
SEGMAX = 16000
MC = 2 * KE
NSET = 2


def _make_segsum_body(with_counts):
    def body(*refs):
        if with_counts:
            ins = refs[:7]
            (tab_u, tab_m, iu_hbm, im_hbm, zb, zs, ones_hbm) = ins
            (out_m, out_u, cnt_m, cnt_u) = refs[7:11]
            rest = refs[11:]
        else:
            ins = refs[:5]
            (tab_u, tab_m, iu_hbm, im_hbm, zb) = ins
            (out_m, out_u) = refs[5:7]
            rest = refs[7:]
        k = 0

        def take(n):
            nonlocal k
            r = rest[k:k + n]
            k += n
            return r

        acc, = take(1)
        cnt = take(1)[0] if with_counts else None
        gi_all, si_all = take(2)
        gi_f = take(2 * NSET)
        si_f = take(2 * NSET)
        r_f = take(2 * NSET)
        gi_w = tuple(gi_f[2 * b:2 * b + 2] for b in range(NSET))
        si_w = tuple(si_f[2 * b:2 * b + 2] for b in range(NSET))
        rw = tuple(r_f[2 * b:2 * b + 2] for b in range(NSET))
        if with_counts:
            ones_v, cv = take(2)
        s_g = take(NSET)
        s_s = take(NSET)
        if with_counts:
            s_c = take(NSET)

        cid = lax.axis_index("c")
        tid = lax.axis_index("s")
        r0 = tid * RPT
        pltpu.sync_copy(zb.at[pl.ds(r0, RPT)], acc.at[pl.ds(r0, RPT)])
        if with_counts:
            pltpu.sync_copy(zs.at[pl.ds(r0, RPT)], cv)
            pltpu.sync_copy(cv, cnt.at[pl.ds(r0, RPT)])
            pltpu.sync_copy(ones_hbm, ones_v)

        def run_dir(tab, gidx_hbm, sidx_hbm, out_a, cnt_o):
            def step(c, b, first):
                if not first:
                    for sub in (0, 1):
                        pltpu.make_async_copy(
                            rw[b][sub], acc.at[si_w[b][sub]], s_s[b]).wait()
                        if with_counts:
                            pltpu.make_async_copy(
                                ones_v, cnt.at[si_w[b][sub]], s_c[b]).wait()
                for sub in (0, 1):
                    giv, siv = gi_w[b][sub], si_w[b][sub]
                    off = c * MC + sub * KE
                    for i in range(KE // 16):
                        giv[pl.ds(i * 16, 16)] = gi_all[pl.ds(off + i * 16, 16)]
                        siv[pl.ds(i * 16, 16)] = si_all[pl.ds(off + i * 16, 16)]
                d = [pltpu.async_copy(tab.at[gi_w[b][sub]], rw[b][sub], s_g[b])
                     for sub in (0, 1)]
                for sub in (0, 1):
                    d[sub].wait()
                    pltpu.async_copy(rw[b][sub], acc.at[si_w[b][sub]],
                                     s_s[b], add=True)
                    if with_counts:
                        pltpu.async_copy(ones_v, cnt.at[si_w[b][sub]],
                                         s_c[b], add=True)

            first_seg = True
            for soff, slen in SEGS:
                base = tid * EPT + soff
                pltpu.sync_copy(gidx_hbm.at[pl.ds(base, slen)],
                                gi_all.at[pl.ds(0, slen)])
                pltpu.sync_copy(sidx_hbm.at[pl.ds(base, slen)],
                                si_all.at[pl.ds(0, slen)])
                if first_seg:
                    plsc.subcore_barrier()
                for b in range(NSET):
                    step(b, b, first_seg)

                def quad(p, carry):
                    for b in range(NSET):
                        step(NSET * p + b, b, False)
                    return carry

                lax.fori_loop(1, slen // MC // NSET, quad, 0)
                first_seg = False
            toff, tlen = TAIL
            if tlen:
                pltpu.sync_copy(gidx_hbm.at[pl.ds(tid * EPT + toff, tlen)],
                                gi_all.at[pl.ds(0, tlen)])
                pltpu.sync_copy(sidx_hbm.at[pl.ds(tid * EPT + toff, tlen)],
                                si_all.at[pl.ds(0, tlen)])
                for i, b in enumerate((0, 1)):
                    step(i, b, False)
            for b in range(NSET):
                for sub in (0, 1):
                    pltpu.make_async_copy(
                        rw[b][sub], acc.at[si_w[b][sub]], s_s[b]).wait()
                    if with_counts:
                        pltpu.make_async_copy(
                            ones_v, cnt.at[si_w[b][sub]], s_c[b]).wait()
            plsc.subcore_barrier()
            pltpu.sync_copy(acc.at[pl.ds(r0, RPT)], out_a.at[pl.ds(r0, RPT)])
            if with_counts:
                pltpu.sync_copy(cnt.at[pl.ds(r0, RPT)], cv)
                pltpu.sync_copy(cv, cnt_o.at[pl.ds(r0, RPT)])

        @pl.when(cid == 0)
        def _():
            run_dir(tab_u, iu_hbm, im_hbm, out_m, cnt_m if with_counts else None)

        @pl.when(cid == 1)
        def _():
            run_dir(tab_m, im_hbm, iu_hbm, out_u, cnt_u if with_counts else None)

    return body


def _seg_scratch(with_counts):
    sc = [
        pltpu.VMEM_SHARED((NPAD, D), _f32),
        pltpu.VMEM_SHARED((NPAD,), _f32) if with_counts else None,
        pltpu.VMEM((SEGMAX,), jnp.int32),
        pltpu.VMEM((SEGMAX,), jnp.int32),
    ]
    sc += [pltpu.VMEM((KE,), jnp.int32) for _ in range(4 * NSET)]
    sc += [pltpu.VMEM((KE, D), _f32) for _ in range(2 * NSET)]
    if with_counts:
        sc += [pltpu.VMEM((KE,), _f32), pltpu.VMEM((RPT,), _f32)]
    sc = [s for s in sc if s is not None]
    nsem = 3 * NSET if with_counts else 2 * NSET
    sc += [pltpu.SemaphoreType.DMA] * nsem
    return sc


_segsum_counts = pl.kernel(
    _make_segsum_body(True),
    out_type=(jax.ShapeDtypeStruct((NPAD, D), _f32),
              jax.ShapeDtypeStruct((NPAD, D), _f32),
              jax.ShapeDtypeStruct((NPAD,), _f32),
              jax.ShapeDtypeStruct((NPAD,), _f32)),
    mesh=_mesh,
    scratch_types=_seg_scratch(True),
)

_segsum = pl.kernel(
    _make_segsum_body(False),
    out_type=(jax.ShapeDtypeStruct((NPAD, D), _f32),
              jax.ShapeDtypeStruct((NPAD, D), _f32)),
    mesh=_mesh,
    scratch_types=_seg_scratch(False),
)


def _label_gather_body(pu_hbm, pm_hbm, row_hbm, col_hbm,
                       gu_o, gm_o, ri_all, ci_all,
                       ru0, ru1, rm0, rm1,
                       sg0, sg1, sw0, sw1):
    ru_w, rm_w = (ru0, ru1), (rm0, rm1)
    s_g, s_w = (sg0, sg1), (sw0, sw1)
    base = _worker_id() * LPW
    pltpu.sync_copy(row_hbm.at[pl.ds(base, LPW)], ri_all)
    pltpu.sync_copy(col_hbm.at[pl.ds(base, LPW)], ci_all)

    def step(c, b, first):
        ru, rm = ru_w[b], rm_w[b]
        if not first:
            off2 = base + (c - 2) * KL
            pltpu.make_async_copy(ru, gu_o.at[pl.ds(off2, KL)], s_w[b]).wait()
            pltpu.make_async_copy(rm, gm_o.at[pl.ds(off2, KL)], s_w[b]).wait()
        loc = c * KL
        d1 = pltpu.async_copy(pu_hbm.at[ri_all.at[pl.ds(loc, KL)]], ru, s_g[b])
        d2 = pltpu.async_copy(pm_hbm.at[ci_all.at[pl.ds(loc, KL)]], rm, s_g[b])
        d1.wait()
        d2.wait()
        off = base + loc
        pltpu.async_copy(ru, gu_o.at[pl.ds(off, KL)], s_w[b])
        pltpu.async_copy(rm, gm_o.at[pl.ds(off, KL)], s_w[b])

    step(0, 0, True)
    step(1, 1, True)

    def pair(p, carry):
        step(2 * p, 0, False)
        step(2 * p + 1, 1, False)
        return carry

    lax.fori_loop(1, NCL // 2, pair, 0)
    for b in (0, 1):
        off2 = base + (NCL - 2 + b) * KL
        pltpu.make_async_copy(ru_w[b], gu_o.at[pl.ds(off2, KL)], s_w[b]).wait()
        pltpu.make_async_copy(rm_w[b], gm_o.at[pl.ds(off2, KL)], s_w[b]).wait()


_label_gather = pl.kernel(
    _label_gather_body,
    out_type=(jax.ShapeDtypeStruct((LPAD, D), _f32),
              jax.ShapeDtypeStruct((LPAD, D), _f32)),
    mesh=_mesh,
    scratch_types=[
        pltpu.VMEM((LPW,), jnp.int32),
        pltpu.VMEM((LPW,), jnp.int32),
        pltpu.VMEM((KL, D), _f32),
        pltpu.VMEM((KL, D), _f32),
        pltpu.VMEM((KL, D), _f32),
        pltpu.VMEM((KL, D), _f32),
    ] + [pltpu.SemaphoreType.DMA] * 4,
)


_BR = 512


def _sage_half(acc, cnt, x, Wl, b, Wr):
    s = acc[...]
    ctot = cnt[...]
    inv = 1.0 / jnp.maximum(ctot, 1.0)
    mean = s * inv[:, None]
    return (jnp.dot(mean, Wl[...], preferred_element_type=jnp.float32)
            + b[...]
            + jnp.dot(x[...], Wr[...], preferred_element_type=jnp.float32))


def _tc_layer1_body(acc_m, cnt_m, xm, WlA, bA, WrA,
                    acc_u, cnt_u, xu, WlB, bB, WrB, hm_o, hu_o):
    hm_o[...] = jnp.maximum(_sage_half(acc_m, cnt_m, xm, WlA, bA, WrA), 0.0)
    hu_o[...] = jnp.maximum(_sage_half(acc_u, cnt_u, xu, WlB, bB, WrB), 0.0)


def _tc_layer2_body(acc_m, cnt_m, xm, WlA, bA, WrA,
                    acc_u, cnt_u, xu, WlB, bB, WrB,
                    WpU, bpU, WpM, pu_o, pm_o):
    zm = _sage_half(acc_m, cnt_m, xm, WlA, bA, WrA)
    zu = _sage_half(acc_u, cnt_u, xu, WlB, bB, WrB)
    pu_o[...] = jnp.dot(zu, WpU[...], preferred_element_type=jnp.float32) + bpU[...]
    pm_o[...] = jnp.dot(zm, WpM[...], preferred_element_type=jnp.float32)


def _acc_spec():
    return pl.BlockSpec((_BR, D), lambda i: (i, 0))


def _cnt_spec():
    return pl.BlockSpec((_BR,), lambda i: (i,))


def _row_spec(w=D):
    return pl.BlockSpec((_BR, w), lambda i: (i, 0))


def _w_spec(w=D):
    return pl.BlockSpec((D, w), lambda i: (0, 0))


def _b_spec(w=D):
    return pl.BlockSpec((1, w), lambda i: (0, 0))


_tc_layer1 = pl.pallas_call(
    _tc_layer1_body,
    grid=(NPAD // _BR,),
    in_specs=[_acc_spec(), _cnt_spec(), _row_spec(), _w_spec(), _b_spec(), _w_spec(),
              _acc_spec(), _cnt_spec(), _row_spec(), _w_spec(), _b_spec(), _w_spec()],
    out_specs=(_row_spec(), _row_spec()),
    out_shape=(jax.ShapeDtypeStruct((NPAD, D), _f32),
               jax.ShapeDtypeStruct((NPAD, D), _f32)),
)

_tc_layer2 = pl.pallas_call(
    _tc_layer2_body,
    grid=(NPAD // _BR,),
    in_specs=[_acc_spec(), _cnt_spec(), _row_spec(), _w_spec(), _b_spec(), _w_spec(),
              _acc_spec(), _cnt_spec(), _row_spec(), _w_spec(), _b_spec(), _w_spec(),
              _w_spec(D), _b_spec(D), _w_spec(D)],
    out_specs=(_row_spec(D), _row_spec(D)),
    out_shape=(jax.ShapeDtypeStruct((NPAD, D), _f32),
               jax.ShapeDtypeStruct((NPAD, D), _f32)),
)


def _head_body(gu, gm, o):
    s = gu[...] + gm[...]
    colid = lax.broadcasted_iota(jnp.int32, s.shape, 1)
    o[...] = jnp.where(colid == 1, jax.nn.softplus(s) + 1e-6, s)


_BL = 1024
_head = pl.pallas_call(
    _head_body,
    grid=(LPAD // _BL,),
    in_specs=[pl.BlockSpec((_BL, D), lambda i: (i, 0)),
              pl.BlockSpec((_BL, D), lambda i: (i, 0))],
    out_specs=pl.BlockSpec((_BL, D), lambda i: (i, 0)),
    out_shape=jax.ShapeDtypeStruct((LPAD, D), _f32),
)


def kernel(x_user, x_movie, edge_index, edge_label_index,
           W1l_um, b1l_um, W1r_um, W1l_mu, b1l_mu, W1r_mu,
           W2l_um, b2l_um, W2r_um, W2l_mu, b2l_mu, W2r_mu, Wlin, blin):
    f32 = jnp.float32
    pad_n = NPAD - N
    xu = jnp.pad(x_user.astype(f32), ((0, pad_n), (0, 0)))
    xm = jnp.pad(x_movie.astype(f32), ((0, pad_n), (0, 0)))
    iu = edge_index[0].astype(jnp.int32)
    im = edge_index[1].astype(jnp.int32)
    row = jnp.pad(edge_label_index[0].astype(jnp.int32), (0, LPAD - L))
    col = jnp.pad(edge_label_index[1].astype(jnp.int32), (0, LPAD - L))

    zb = jnp.zeros((NPAD, D), f32)
    zs = jnp.zeros((NPAD,), f32)
    ones = jnp.ones((KE,), f32)

    acc_m, acc_u, cnt_m, cnt_u = _segsum_counts(xu, xm, iu, im, zb, zs, ones)

    b1um = b1l_um.reshape(1, D).astype(f32)
    b1mu = b1l_mu.reshape(1, D).astype(f32)
    h_movie, h_user = _tc_layer1(acc_m, cnt_m, xm, W1l_um, b1um, W1r_um,
                                 acc_u, cnt_u, xu, W1l_mu, b1mu, W1r_mu)

    acc2_m, acc2_u = _segsum(h_user, h_movie, iu, im, zb)

    WpU = jnp.zeros((D, D), f32).at[:, 0:2].set(Wlin[:D].astype(f32))
    WpM = jnp.zeros((D, D), f32).at[:, 0:2].set(Wlin[D:].astype(f32))
    bp = jnp.zeros((1, D), f32).at[0, 0:2].set(blin.astype(f32))
    b2um = b2l_um.reshape(1, D).astype(f32)
    b2mu = b2l_mu.reshape(1, D).astype(f32)
    p_user, p_movie = _tc_layer2(acc2_m, cnt_m, h_movie, W2l_um, b2um, W2r_um,
                                 acc2_u, cnt_u, h_user, W2l_mu, b2mu, W2r_mu,
                                 WpU, bp, WpM)

    gu, gm = _label_gather(p_user, p_movie, row, col)
    out = _head(gu, gm)
    return out[:L, 0], out[:L, 1]


# speedup vs baseline: 1.0069x; 1.0049x over previous
import jax
import jax.numpy as jnp
from jax import lax
from jax.experimental import pallas as pl
from jax.experimental.pallas import tpu as pltpu
from jax.experimental.pallas import tpu_sc as plsc

NC, NS = 2, 16
NW = NC * NS
N = 5000
NPAD = 5120
RPT = NPAD // NS
E = 640000
EPW = E // NW
KE = 80
NCH = EPW // KE
L = 100000
LPAD = 100352
LPW = LPAD // NW
KL = 56
NCL = LPW // KL
NBL = 4
D = 128
CW = 16
_f32 = jnp.float32

_mesh = plsc.VectorSubcoreMesh(
    core_axis_name="c", subcore_axis_name="s", num_cores=NC, num_subcores=NS)


def _worker_id():
    return lax.axis_index("s") * NC + lax.axis_index("c")


EPT = E // NS
SEGS = ((0, 16000), (16000, 16000), (32000, 8000))
TAIL = (0, 0)
SEGMAX = 16000
MC = 2 * KE
NSET = 2


def _make_segsum_body(with_counts):
    def body(*refs):
        if with_counts:
            ins = refs[:7]
            (tab_u, tab_m, iu_hbm, im_hbm, zb, zs, ones_hbm) = ins
            (out_m, out_u, cnt_m, cnt_u) = refs[7:11]
            rest = refs[11:]
        else:
            ins = refs[:5]
            (tab_u, tab_m, iu_hbm, im_hbm, zb) = ins
            (out_m, out_u) = refs[5:7]
            rest = refs[7:]
        k = 0

        def take(n):
            nonlocal k
            r = rest[k:k + n]
            k += n
            return r

        acc, = take(1)
        cnt = take(1)[0] if with_counts else None
        gi_all, si_all = take(2)
        gi_f = take(2 * NSET)
        si_f = take(2 * NSET)
        r_f = take(2 * NSET)
        gi_w = tuple(gi_f[2 * b:2 * b + 2] for b in range(NSET))
        si_w = tuple(si_f[2 * b:2 * b + 2] for b in range(NSET))
        rw = tuple(r_f[2 * b:2 * b + 2] for b in range(NSET))
        if with_counts:
            ones_v, cv = take(2)
        s_g = take(NSET)
        s_s = take(NSET)
        if with_counts:
            s_c = take(NSET)

        cid = lax.axis_index("c")
        tid = lax.axis_index("s")
        r0 = tid * RPT
        pltpu.sync_copy(zb.at[pl.ds(r0, RPT)], acc.at[pl.ds(r0, RPT)])
        if with_counts:
            pltpu.sync_copy(zs.at[pl.ds(r0, RPT)], cv)
            pltpu.sync_copy(cv, cnt.at[pl.ds(r0, RPT)])
            pltpu.sync_copy(ones_hbm, ones_v)

        def run_dir(tab, gidx_hbm, sidx_hbm, out_a, cnt_o):
            def step(c, b, first):
                if not first:
                    for sub in (0, 1):
                        pltpu.make_async_copy(
                            rw[b][sub], acc.at[si_w[b][sub]], s_s[b]).wait()
                        if with_counts:
                            pltpu.make_async_copy(
                                ones_v, cnt.at[si_w[b][sub]], s_c[b]).wait()
                for sub in (0, 1):
                    giv, siv = gi_w[b][sub], si_w[b][sub]
                    off = c * MC + sub * KE
                    for i in range(KE // 16):
                        giv[pl.ds(i * 16, 16)] = gi_all[pl.ds(off + i * 16, 16)]
                        siv[pl.ds(i * 16, 16)] = si_all[pl.ds(off + i * 16, 16)]
                d = [pltpu.async_copy(tab.at[gi_w[b][sub]], rw[b][sub], s_g[b])
                     for sub in (0, 1)]
                for sub in (0, 1):
                    d[sub].wait()
                    pltpu.async_copy(rw[b][sub], acc.at[si_w[b][sub]],
                                     s_s[b], add=True)
                    if with_counts:
                        pltpu.async_copy(ones_v, cnt.at[si_w[b][sub]],
                                         s_c[b], add=True)

            first_seg = True
            for soff, slen in SEGS:
                base = tid * EPT + soff
                pltpu.sync_copy(gidx_hbm.at[pl.ds(base, slen)],
                                gi_all.at[pl.ds(0, slen)])
                pltpu.sync_copy(sidx_hbm.at[pl.ds(base, slen)],
                                si_all.at[pl.ds(0, slen)])
                if first_seg:
                    plsc.subcore_barrier()
                for b in range(NSET):
                    step(b, b, first_seg)

                def quad(p, carry):
                    for b in range(NSET):
                        step(NSET * p + b, b, False)
                    return carry

                lax.fori_loop(1, slen // MC // NSET, quad, 0)
                first_seg = False
            toff, tlen = TAIL
            if tlen:
                pltpu.sync_copy(gidx_hbm.at[pl.ds(tid * EPT + toff, tlen)],
                                gi_all.at[pl.ds(0, tlen)])
                pltpu.sync_copy(sidx_hbm.at[pl.ds(tid * EPT + toff, tlen)],
                                si_all.at[pl.ds(0, tlen)])
                for i, b in enumerate((0, 1)):
                    step(i, b, False)
            for b in range(NSET):
                for sub in (0, 1):
                    pltpu.make_async_copy(
                        rw[b][sub], acc.at[si_w[b][sub]], s_s[b]).wait()
                    if with_counts:
                        pltpu.make_async_copy(
                            ones_v, cnt.at[si_w[b][sub]], s_c[b]).wait()
            plsc.subcore_barrier()
            pltpu.sync_copy(acc.at[pl.ds(r0, RPT)], out_a.at[pl.ds(r0, RPT)])
            if with_counts:
                pltpu.sync_copy(cnt.at[pl.ds(r0, RPT)], cv)
                pltpu.sync_copy(cv, cnt_o.at[pl.ds(r0, RPT)])

        @pl.when(cid == 0)
        def _():
            run_dir(tab_u, iu_hbm, im_hbm, out_m, cnt_m if with_counts else None)

        @pl.when(cid == 1)
        def _():
            run_dir(tab_m, im_hbm, iu_hbm, out_u, cnt_u if with_counts else None)

    return body


def _seg_scratch(with_counts):
    sc = [
        pltpu.VMEM_SHARED((NPAD, D), _f32),
        pltpu.VMEM_SHARED((NPAD,), _f32) if with_counts else None,
        pltpu.VMEM((SEGMAX,), jnp.int32),
        pltpu.VMEM((SEGMAX,), jnp.int32),
    ]
    sc += [pltpu.VMEM((KE,), jnp.int32) for _ in range(4 * NSET)]
    sc += [pltpu.VMEM((KE, D), _f32) for _ in range(2 * NSET)]
    if with_counts:
        sc += [pltpu.VMEM((KE,), _f32), pltpu.VMEM((RPT,), _f32)]
    sc = [s for s in sc if s is not None]
    nsem = 3 * NSET if with_counts else 2 * NSET
    sc += [pltpu.SemaphoreType.DMA] * nsem
    return sc


_segsum_counts = pl.kernel(
    _make_segsum_body(True),
    out_type=(jax.ShapeDtypeStruct((NPAD, D), _f32),
              jax.ShapeDtypeStruct((NPAD, D), _f32),
              jax.ShapeDtypeStruct((NPAD,), _f32),
              jax.ShapeDtypeStruct((NPAD,), _f32)),
    mesh=_mesh,
    scratch_types=_seg_scratch(True),
)

_segsum = pl.kernel(
    _make_segsum_body(False),
    out_type=(jax.ShapeDtypeStruct((NPAD, D), _f32),
              jax.ShapeDtypeStruct((NPAD, D), _f32)),
    mesh=_mesh,
    scratch_types=_seg_scratch(False),
)


def _label_gather_body(pu_hbm, pm_hbm, row_hbm, col_hbm,
                       gu_o, gm_o, ri_all, ci_all, *rest):
    ru_w = rest[0:NBL]
    rm_w = rest[NBL:2 * NBL]
    s_g = rest[2 * NBL:3 * NBL]
    s_w = rest[3 * NBL:4 * NBL]
    base = _worker_id() * LPW
    pltpu.sync_copy(row_hbm.at[pl.ds(base, LPW)], ri_all)
    pltpu.sync_copy(col_hbm.at[pl.ds(base, LPW)], ci_all)

    def issue_g(c, b):
        loc = c * KL
        pltpu.async_copy(pu_hbm.at[ri_all.at[pl.ds(loc, KL)]], ru_w[b], s_g[b])
        pltpu.async_copy(pm_hbm.at[ci_all.at[pl.ds(loc, KL)]], rm_w[b], s_g[b])

    def step(c, b, first, last):
        ru, rm = ru_w[b], rm_w[b]
        loc = c * KL
        pltpu.make_async_copy(pu_hbm.at[ri_all.at[pl.ds(loc, KL)]],
                              ru, s_g[b]).wait()
        pltpu.make_async_copy(pm_hbm.at[ci_all.at[pl.ds(loc, KL)]],
                              rm, s_g[b]).wait()
        off = base + loc
        pltpu.async_copy(ru, gu_o.at[pl.ds(off, KL)], s_w[b])
        pltpu.async_copy(rm, gm_o.at[pl.ds(off, KL)], s_w[b])
        if not last:
            b2 = (b + 2) % NBL
            if not first:
                off2 = base + (c - 2) * KL
                pltpu.make_async_copy(ru_w[b2], gu_o.at[pl.ds(off2, KL)],
                                      s_w[b2]).wait()
                pltpu.make_async_copy(rm_w[b2], gm_o.at[pl.ds(off2, KL)],
                                      s_w[b2]).wait()
            issue_g(c + 2, b2)

    issue_g(0, 0)
    issue_g(1, 1)
    for c in range(NBL):
        step(c, c % NBL, c < 2, False)

    def quad(p, carry):
        for j in range(NBL):
            step(NBL * p + j, j, False, False)
        return carry

    lax.fori_loop(1, NCL // NBL - 1, quad, 0)
    for c in range(NCL - NBL, NCL):
        step(c, c % NBL, False, c >= NCL - 2)
    for c in range(NCL - NBL, NCL):
        b = c % NBL
        off2 = base + c * KL
        pltpu.make_async_copy(ru_w[b], gu_o.at[pl.ds(off2, KL)], s_w[b]).wait()
        pltpu.make_async_copy(rm_w[b], gm_o.at[pl.ds(off2, KL)], s_w[b]).wait()


_label_gather = pl.kernel(
    _label_gather_body,
    out_type=(jax.ShapeDtypeStruct((LPAD, D), _f32),
              jax.ShapeDtypeStruct((LPAD, D), _f32)),
    mesh=_mesh,
    scratch_types=[
        pltpu.VMEM((LPW,), jnp.int32),
        pltpu.VMEM((LPW,), jnp.int32),
    ] + [pltpu.VMEM((KL, D), _f32) for _ in range(2 * NBL)]
      + [pltpu.SemaphoreType.DMA] * (2 * NBL),
)


_BR = 512


def _sage_half(acc, cnt, x, Wl, b, Wr):
    s = acc[...]
    ctot = cnt[...]
    inv = 1.0 / jnp.maximum(ctot, 1.0)
    mean = s * inv[:, None]
    return (jnp.dot(mean, Wl[...], preferred_element_type=jnp.float32)
            + b[...]
            + jnp.dot(x[...], Wr[...], preferred_element_type=jnp.float32))


def _tc_layer1_body(acc_m, cnt_m, xm, WlA, bA, WrA,
                    acc_u, cnt_u, xu, WlB, bB, WrB, hm_o, hu_o):
    hm_o[...] = jnp.maximum(_sage_half(acc_m, cnt_m, xm, WlA, bA, WrA), 0.0)
    hu_o[...] = jnp.maximum(_sage_half(acc_u, cnt_u, xu, WlB, bB, WrB), 0.0)


def _tc_layer2_body(acc_m, cnt_m, xm, WlA, bA, WrA,
                    acc_u, cnt_u, xu, WlB, bB, WrB,
                    WpU, bpU, WpM, pu_o, pm_o):
    zm = _sage_half(acc_m, cnt_m, xm, WlA, bA, WrA)
    zu = _sage_half(acc_u, cnt_u, xu, WlB, bB, WrB)
    pu_o[...] = jnp.dot(zu, WpU[...], preferred_element_type=jnp.float32) + bpU[...]
    pm_o[...] = jnp.dot(zm, WpM[...], preferred_element_type=jnp.float32)


def _acc_spec():
    return pl.BlockSpec((_BR, D), lambda i: (i, 0))


def _cnt_spec():
    return pl.BlockSpec((_BR,), lambda i: (i,))


def _row_spec(w=D):
    return pl.BlockSpec((_BR, w), lambda i: (i, 0))


def _w_spec(w=D):
    return pl.BlockSpec((D, w), lambda i: (0, 0))


def _b_spec(w=D):
    return pl.BlockSpec((1, w), lambda i: (0, 0))


_tc_layer1 = pl.pallas_call(
    _tc_layer1_body,
    grid=(NPAD // _BR,),
    in_specs=[_acc_spec(), _cnt_spec(), _row_spec(), _w_spec(), _b_spec(), _w_spec(),
              _acc_spec(), _cnt_spec(), _row_spec(), _w_spec(), _b_spec(), _w_spec()],
    out_specs=(_row_spec(), _row_spec()),
    out_shape=(jax.ShapeDtypeStruct((NPAD, D), _f32),
               jax.ShapeDtypeStruct((NPAD, D), _f32)),
)

_tc_layer2 = pl.pallas_call(
    _tc_layer2_body,
    grid=(NPAD // _BR,),
    in_specs=[_acc_spec(), _cnt_spec(), _row_spec(), _w_spec(), _b_spec(), _w_spec(),
              _acc_spec(), _cnt_spec(), _row_spec(), _w_spec(), _b_spec(), _w_spec(),
              _w_spec(D), _b_spec(D), _w_spec(D)],
    out_specs=(_row_spec(D), _row_spec(D)),
    out_shape=(jax.ShapeDtypeStruct((NPAD, D), _f32),
               jax.ShapeDtypeStruct((NPAD, D), _f32)),
)


def _head_body(gu, gm, o):
    s = gu[...] + gm[...]
    colid = lax.broadcasted_iota(jnp.int32, s.shape, 1)
    o[...] = jnp.where(colid == 1, jax.nn.softplus(s) + 1e-6, s)


_BL = 1024
_head = pl.pallas_call(
    _head_body,
    grid=(LPAD // _BL,),
    in_specs=[pl.BlockSpec((_BL, D), lambda i: (i, 0)),
              pl.BlockSpec((_BL, D), lambda i: (i, 0))],
    out_specs=pl.BlockSpec((_BL, D), lambda i: (i, 0)),
    out_shape=jax.ShapeDtypeStruct((LPAD, D), _f32),
)


def kernel(x_user, x_movie, edge_index, edge_label_index,
           W1l_um, b1l_um, W1r_um, W1l_mu, b1l_mu, W1r_mu,
           W2l_um, b2l_um, W2r_um, W2l_mu, b2l_mu, W2r_mu, Wlin, blin):
    f32 = jnp.float32
    pad_n = NPAD - N
    xu = jnp.pad(x_user.astype(f32), ((0, pad_n), (0, 0)))
    xm = jnp.pad(x_movie.astype(f32), ((0, pad_n), (0, 0)))
    iu = edge_index[0].astype(jnp.int32)
    im = edge_index[1].astype(jnp.int32)
    row = jnp.pad(edge_label_index[0].astype(jnp.int32), (0, LPAD - L))
    col = jnp.pad(edge_label_index[1].astype(jnp.int32), (0, LPAD - L))

    zb = jnp.zeros((NPAD, D), f32)
    zs = jnp.zeros((NPAD,), f32)
    ones = jnp.ones((KE,), f32)

    acc_m, acc_u, cnt_m, cnt_u = _segsum_counts(xu, xm, iu, im, zb, zs, ones)

    b1um = b1l_um.reshape(1, D).astype(f32)
    b1mu = b1l_mu.reshape(1, D).astype(f32)
    h_movie, h_user = _tc_layer1(acc_m, cnt_m, xm, W1l_um, b1um, W1r_um,
                                 acc_u, cnt_u, xu, W1l_mu, b1mu, W1r_mu)

    acc2_m, acc2_u = _segsum(h_user, h_movie, iu, im, zb)

    WpU = jnp.zeros((D, D), f32).at[:, 0:2].set(Wlin[:D].astype(f32))
    WpM = jnp.zeros((D, D), f32).at[:, 0:2].set(Wlin[D:].astype(f32))
    bp = jnp.zeros((1, D), f32).at[0, 0:2].set(blin.astype(f32))
    b2um = b2l_um.reshape(1, D).astype(f32)
    b2mu = b2l_mu.reshape(1, D).astype(f32)
    p_user, p_movie = _tc_layer2(acc2_m, cnt_m, h_movie, W2l_um, b2um, W2r_um,
                                 acc2_u, cnt_u, h_user, W2l_mu, b2mu, W2r_mu,
                                 WpU, bp, WpM)

    gu, gm = _label_gather(p_user, p_movie, row, col)
    out = _head(gu, gm)
    return out[:L, 0], out[:L, 1]
